# baseline (speedup 1.0000x reference)
import functools

import jax
import jax.numpy as jnp
from jax import lax
from jax.experimental import pallas as pl
from jax.experimental.pallas import tpu as pltpu
from jax.experimental.pallas import tpu_sc as plsc

V = 1000000
B = 16384
D = 64
LANES = 16
GPD = V // LANES
NC = 2
NS = 16
NW = NC * NS
BPW = B // NW
CHUNK = 128
NCH = BPW // CHUNK
WD = 4
WCH = WD * NCH
NWAVE = D // WD
VPC = CHUNK // LANES


def _sc_embed_lookup(X, tab_g, sh_bcast):
    mesh = plsc.VectorSubcoreMesh(core_axis_name="c", subcore_axis_name="s")

    @functools.partial(
        pl.kernel,
        mesh=mesh,
        out_type=jax.ShapeDtypeStruct((D, B), jnp.float32),
        compiler_params=pltpu.CompilerParams(
            use_tc_tiling_on_sc=False, needs_layout_passes=False
        ),
        scratch_types=[
            pltpu.VMEM((NCH, CHUNK), jnp.int32),
            pltpu.VMEM((2, WCH, CHUNK), jnp.int32),
            pltpu.VMEM((2, WCH, CHUNK, LANES), jnp.float32),
            pltpu.VMEM((D, BPW), jnp.float32),
            pltpu.VMEM((D, LANES), jnp.float32),
            pltpu.SemaphoreType.DMA,
            pltpu.SemaphoreType.DMA,
        ],
    )
    def body(x_hbm, tab_hbm, sh_hbm, out_hbm, xa, ib, stg, rows, sh_v, gsem, osem):
        wid = lax.axis_index("s") * NC + lax.axis_index("c")
        base = wid * BPW

        pltpu.sync_copy(sh_hbm, sh_v)
        for j in range(NCH):
            pltpu.sync_copy(
                x_hbm.at[pl.ds(base + j * CHUNK, CHUNK)], xa.at[j]
            )

        def build_wave(w, buf):
            for c in range(WCH):
                dd = WD * w + c // NCH
                j = c % NCH
                off = dd * jnp.int32(GPD)
                for k in range(VPC):
                    sl = pl.ds(k * LANES, LANES)
                    ib[buf, c, sl] = (xa[j, sl] >> 4) + off

        def fire_wave(buf):
            for c in range(WCH):
                pltpu.async_copy(
                    tab_hbm.at[ib.at[buf, c]], stg.at[buf, c], gsem
                )

        def drain_wave(buf):
            for c in range(WCH):
                pltpu.make_async_copy(
                    tab_hbm.at[pl.ds(0, CHUNK)], stg.at[buf, c], gsem
                ).wait()

        def extract_wave(w, buf):
            iot = lax.iota(jnp.int32, LANES)
            for c in range(WCH):
                dd = WD * w + c // NCH
                j = c % NCH
                sv = sh_v[dd, :]
                for q in range(VPC):
                    sl = pl.ds(q * LANES, LANES)
                    lanes = xa[j, sl] & 15
                    v = plsc.load_gather(
                        stg.at[buf, c], [q * LANES + iot, lanes]
                    )
                    rows[dd, pl.ds(j * CHUNK + q * LANES, LANES)] = v + sv

        def out_wave(w):
            pltpu.async_copy(
                rows.at[pl.ds(WD * w, WD)],
                out_hbm.at[pl.ds(WD * w, WD), pl.ds(base, BPW)],
                osem,
            )

        z = jnp.int32(0)
        build_wave(z, z)
        fire_wave(z)

        def wave_body(g, carry):
            bn = (g + 1) & 1
            bc = g & 1
            build_wave(g + 1, bn)
            fire_wave(bn)
            drain_wave(bc)
            extract_wave(g, bc)
            out_wave(g)
            return carry

        lax.fori_loop(0, NWAVE - 1, wave_body, 0)

        last = jnp.int32(NWAVE - 1)
        lb = jnp.int32((NWAVE - 1) & 1)
        drain_wave(lb)
        extract_wave(last, lb)
        out_wave(last)
        pltpu.make_async_copy(
            out_hbm.at[pl.ds(0, D), pl.ds(base, BPW)], rows, osem
        ).wait()

    return body(X, tab_g, sh_bcast)


def kernel(X, embed_table, shared_embed):
    tab_g = embed_table.T.reshape(D * V // LANES, LANES)
    sh_bcast = jnp.broadcast_to(shared_embed.reshape(D, 1), (D, LANES))
    out_t = _sc_embed_lookup(X, tab_g, sh_bcast)
    return out_t.T


# speedup vs baseline: 2.0984x; 2.0984x over previous
import functools

import jax
import jax.numpy as jnp
from jax import lax
from jax.experimental import pallas as pl
from jax.experimental.pallas import tpu as pltpu
from jax.experimental.pallas import tpu_sc as plsc

V = 1000000
B = 16384
D = 64
NSL = 4
DS = D // NSL
NC = 2
NS = 16
NW = NC * NS
BPW = B // NW
CHUNK = 128
NCH = BPW // CHUNK
LANES = 16


def _sc_embed_lookup(X, t0, t1, t2, t3, shared_flat):
    mesh = plsc.VectorSubcoreMesh(core_axis_name="c", subcore_axis_name="s")

    @functools.partial(
        pl.kernel,
        mesh=mesh,
        out_type=jax.ShapeDtypeStruct((B, D), jnp.float32),
        compiler_params=pltpu.CompilerParams(
            use_tc_tiling_on_sc=False, needs_layout_passes=False
        ),
        scratch_types=[
            pltpu.VMEM((NCH, CHUNK), jnp.int32),
            [pltpu.VMEM((BPW, DS), jnp.float32) for _ in range(NSL)],
            pltpu.VMEM((D,), jnp.float32),
            pltpu.SemaphoreType.DMA,
            pltpu.SemaphoreType.DMA,
        ],
    )
    def body(x_hbm, *refs):
        tabs = refs[:NSL]
        sh_hbm = refs[NSL]
        out_hbm = refs[NSL + 1]
        idx_v = refs[NSL + 2]
        rows = refs[NSL + 3]
        sh_v = refs[NSL + 4]
        gsem = refs[NSL + 5]
        osem = refs[NSL + 6]

        wid = lax.axis_index("s") * NC + lax.axis_index("c")
        base = wid * BPW

        pltpu.sync_copy(sh_hbm, sh_v)
        for j in range(NCH):
            pltpu.sync_copy(
                x_hbm.at[pl.ds(base + j * CHUNK, CHUNK)], idx_v.at[j]
            )
        for h in range(NSL):
            for j in range(NCH):
                pltpu.async_copy(
                    tabs[h].at[idx_v.at[j]],
                    rows[h].at[pl.ds(j * CHUNK, CHUNK)],
                    gsem,
                )
        for h in range(NSL):
            for j in range(NCH):
                pltpu.make_async_copy(
                    tabs[h].at[pl.ds(0, CHUNK)],
                    rows[h].at[pl.ds(j * CHUNK, CHUNK)],
                    gsem,
                ).wait()

        svs = [sh_v[pl.ds(h * LANES, LANES)] for h in range(NSL)]

        def add_row(i, carry):
            for h in range(NSL):
                plsc.addupdate(rows[h].at[i, :], svs[h])
            return carry

        lax.fori_loop(0, BPW, add_row, 0)

        for h in range(NSL):
            pltpu.async_copy(
                rows[h],
                out_hbm.at[pl.ds(base, BPW), pl.ds(h * DS, DS)],
                osem,
            )
        for h in range(NSL):
            pltpu.make_async_copy(
                out_hbm.at[pl.ds(base, BPW), pl.ds(h * DS, DS)],
                rows[h],
                osem,
            ).wait()

    return body(X, t0, t1, t2, t3, shared_flat)


def kernel(X, embed_table, shared_embed):
    ts = [embed_table[:, h * DS : (h + 1) * DS] for h in range(NSL)]
    return _sc_embed_lookup(X, *ts, shared_embed.reshape(D))


# speedup vs baseline: 8.0141x; 3.8192x over previous
import functools

import jax
import jax.numpy as jnp
from jax import lax
from jax.experimental import pallas as pl
from jax.experimental.pallas import tpu as pltpu
from jax.experimental.pallas import tpu_sc as plsc

V = 1000000
B = 16384
D = 64
VR = V // 2
NC = 2
NS = 16
NW = NC * NS
BPW = B // NW
CHUNK = 128
NCH = BPW // CHUNK
LANES = 16
VPE = D // LANES


def _sc_embed_lookup(X, tab_r, shared_flat):
    mesh = plsc.VectorSubcoreMesh(core_axis_name="c", subcore_axis_name="s")

    @functools.partial(
        pl.kernel,
        mesh=mesh,
        out_type=jax.ShapeDtypeStruct((B, D), jnp.float32),
        compiler_params=pltpu.CompilerParams(
            use_tc_tiling_on_sc=True, needs_layout_passes=False
        ),
        scratch_types=[
            pltpu.VMEM((NCH, CHUNK), jnp.int32),
            pltpu.VMEM((NCH, CHUNK), jnp.int32),
            [pltpu.VMEM((CHUNK, 2 * D), jnp.float32) for _ in range(2)],
            [pltpu.VMEM((CHUNK, D), jnp.float32) for _ in range(2)],
            pltpu.VMEM((D,), jnp.float32),
            pltpu.SemaphoreType.DMA,
            pltpu.SemaphoreType.DMA,
        ],
    )
    def body(x_hbm, tab_hbm, sh_hbm, out_hbm, idx_v, xr, stg, ob, sh_v, gsem, osem):
        wid = lax.axis_index("s") * NC + lax.axis_index("c")
        base = wid * BPW

        pltpu.sync_copy(sh_hbm, sh_v)
        for j in range(NCH):
            pltpu.sync_copy(
                x_hbm.at[pl.ds(base + j * CHUNK, CHUNK)], xr.at[j]
            )
        for j in range(NCH):
            for k in range(CHUNK // LANES):
                sl = pl.ds(k * LANES, LANES)
                idx_v[j, sl] = xr[j, sl] >> 1

        svs = [sh_v[pl.ds(k * LANES, LANES)] for k in range(VPE)]

        def fire(w):
            pltpu.async_copy(tab_hbm.at[idx_v.at[w]], stg[w & 1], gsem)

        def drain(w):
            pltpu.make_async_copy(
                tab_hbm.at[pl.ds(0, CHUNK)], stg[w & 1], gsem
            ).wait()

        def extract(w):
            sb = stg[w & 1]
            od = ob[w & 1]

            def group(q, carry):
                xvec = xr[w, pl.ds(q * LANES, LANES)]
                for l in range(LANES):
                    pv = jnp.full((LANES,), xvec[l] & 1, jnp.int32) > 0
                    e = q * LANES + l
                    for k in range(VPE):
                        lo = sb[e, pl.ds(k * LANES, LANES)]
                        hi = sb[e, pl.ds(D + k * LANES, LANES)]
                        od[e, pl.ds(k * LANES, LANES)] = (
                            jnp.where(pv, hi, lo) + svs[k]
                        )
                return carry

            lax.fori_loop(0, CHUNK // LANES, group, 0)

        def out_wave(w):
            pltpu.async_copy(
                ob[w & 1],
                out_hbm.at[pl.ds(base + w * CHUNK, CHUNK)],
                osem,
            )

        def out_drain(w):
            pltpu.make_async_copy(
                out_hbm.at[pl.ds(base + w * CHUNK, CHUNK)],
                ob[w & 1],
                osem,
            ).wait()

        fire(0)
        for w in range(NCH):
            if w + 1 < NCH:
                fire(w + 1)
            drain(w)
            if w >= 2:
                out_drain(w - 2)
            extract(w)
            out_wave(w)
        out_drain(NCH - 2)
        out_drain(NCH - 1)

    return body(X, tab_r, shared_flat)


def kernel(X, embed_table, shared_embed):
    tab_r = embed_table.reshape(VR, 2 * D)
    return _sc_embed_lookup(X, tab_r, shared_embed.reshape(D))


# speedup vs baseline: 18.5246x; 2.3115x over previous
import functools

import jax
import jax.numpy as jnp
from jax import lax
from jax.experimental import pallas as pl
from jax.experimental.pallas import tpu as pltpu
from jax.experimental.pallas import tpu_sc as plsc

V = 1000000
B = 16384
D = 64
NC = 2
NS = 16
NW = NC * NS
BPW = B // NW
GRP = 16
NGRP = BPW // GRP
SUB = 8
LANES = 16
VPE = D // LANES


def _sc_embed_lookup(X, tab_t, shared_flat):
    mesh = plsc.VectorSubcoreMesh(core_axis_name="c", subcore_axis_name="s")

    @functools.partial(
        pl.kernel,
        mesh=mesh,
        out_type=jax.ShapeDtypeStruct((B // 2, 2 * D), jnp.float32),
        compiler_params=pltpu.CompilerParams(
            use_tc_tiling_on_sc=True, needs_layout_passes=False
        ),
        scratch_types=[
            pltpu.VMEM((BPW,), jnp.int32),
            pltpu.VMEM((SUB, D, 2 * D), jnp.float32),
            pltpu.VMEM((BPW // 2, 2 * D), jnp.float32),
            pltpu.VMEM((D,), jnp.float32),
            pltpu.SemaphoreType.DMA,
        ],
    )
    def body(x_hbm, tab_hbm, sh_hbm, out_hbm, xr, stg, ob, sh_v, gsem):
        wid = lax.axis_index("s") * NC + lax.axis_index("c")
        base = wid * BPW

        pltpu.sync_copy(sh_hbm, sh_v)
        for j in range(4):
            pltpu.sync_copy(
                x_hbm.at[pl.ds(base + j * 128, 128)], xr.at[pl.ds(j * 128, 128)]
            )

        svs = [sh_v[pl.ds(k * LANES, LANES)] for k in range(VPE)]
        iot = lax.iota(jnp.int32, LANES)

        def group(g, carry):
            xvec = xr[pl.ds(g * GRP, GRP)]
            for s in range(GRP // SUB):
                for l in range(SUB):
                    x = xvec[s * SUB + l]
                    col = pl.multiple_of((x >> 7) << 7, 2 * D)
                    pltpu.async_copy(
                        tab_hbm.at[pl.ds(0, D), pl.ds(col, 2 * D)],
                        stg.at[l],
                        gsem,
                    )
                for l in range(SUB):
                    pltpu.make_async_copy(
                        tab_hbm.at[pl.ds(0, D), pl.ds(0, 2 * D)],
                        stg.at[l],
                        gsem,
                    ).wait()
                for l in range(SUB):
                    x = xvec[s * SUB + l]
                    cvec = jnp.full((LANES,), x & 127, jnp.int32)
                    eh = s * SUB + l
                    row = g * (GRP // 2) + eh // 2
                    lane0 = (eh % 2) * D
                    for k in range(VPE):
                        v = plsc.load_gather(
                            stg.at[l], [k * LANES + iot, cvec]
                        )
                        ob[row, pl.ds(lane0 + k * LANES, LANES)] = v + svs[k]
            return carry

        lax.fori_loop(0, NGRP, group, 0)
        pltpu.sync_copy(ob, out_hbm.at[pl.ds(wid * (BPW // 2), BPW // 2)])

    return body(X, tab_t, shared_flat)


def kernel(X, embed_table, shared_embed):
    out_p = _sc_embed_lookup(X, embed_table.T, shared_embed.reshape(D))
    return out_p.reshape(B, D)


# speedup vs baseline: 23.8988x; 1.2901x over previous
import functools

import jax
import jax.numpy as jnp
from jax import lax
from jax.experimental import pallas as pl
from jax.experimental.pallas import tpu as pltpu
from jax.experimental.pallas import tpu_sc as plsc

V = 1000000
B = 16384
D = 64
NC = 2
NS = 16
NW = NC * NS
BPW = B // NW
GRP = 16
NGRP = BPW // GRP
SUB = 4
LANES = 16
VPE = D // LANES


def _sc_embed_lookup(X, tab_t, shared_flat):
    mesh = plsc.VectorSubcoreMesh(core_axis_name="c", subcore_axis_name="s")

    @functools.partial(
        pl.kernel,
        mesh=mesh,
        out_type=jax.ShapeDtypeStruct((B // 2, 2 * D), jnp.float32),
        compiler_params=pltpu.CompilerParams(
            use_tc_tiling_on_sc=True, needs_layout_passes=False
        ),
        scratch_types=[
            pltpu.VMEM((BPW,), jnp.int32),
            [pltpu.VMEM((SUB, D, 2 * D), jnp.float32) for _ in range(2)],
            pltpu.VMEM((BPW // 2, 2 * D), jnp.float32),
            pltpu.VMEM((D,), jnp.float32),
            pltpu.SemaphoreType.DMA,
        ],
    )
    def body(x_hbm, tab_hbm, sh_hbm, out_hbm, xr, stg, ob, sh_v, gsem):
        wid = lax.axis_index("s") * NC + lax.axis_index("c")
        base = wid * BPW

        pltpu.sync_copy(sh_hbm, sh_v)
        for j in range(4):
            pltpu.sync_copy(
                x_hbm.at[pl.ds(base + j * 128, 128)], xr.at[pl.ds(j * 128, 128)]
            )

        svs = [sh_v[pl.ds(k * LANES, LANES)] for k in range(VPE)]
        iot = lax.iota(jnp.int32, LANES)

        def fire(xv, lb, buf):
            for l in range(SUB):
                x = xv[lb + l]
                col = pl.multiple_of((x >> 7) << 7, 2 * D)
                pltpu.async_copy(
                    tab_hbm.at[pl.ds(0, D), pl.ds(col, 2 * D)],
                    stg[buf].at[l],
                    gsem,
                )

        def drain(buf):
            for l in range(SUB):
                pltpu.make_async_copy(
                    tab_hbm.at[pl.ds(0, D), pl.ds(0, 2 * D)],
                    stg[buf].at[l],
                    gsem,
                ).wait()

        def extract(xv, lb, buf, g):
            for l in range(SUB):
                x = xv[lb + l]
                cvec = jnp.full((LANES,), x & 127, jnp.int32)
                eh = lb + l
                row = g * (GRP // 2) + eh // 2
                lane0 = (eh % 2) * D
                for k in range(VPE):
                    v = plsc.load_gather(
                        stg[buf].at[l], [k * LANES + iot, cvec]
                    )
                    ob[row, pl.ds(lane0 + k * LANES, LANES)] = v + svs[k]

        NSW = GRP // SUB

        def group(g, carry):
            xv = xr[pl.ds(g * GRP, GRP)]
            for s in range(NSW - 1):
                fire(xv, (s + 1) * SUB, (s + 1) & 1)
                drain(s & 1)
                extract(xv, s * SUB, s & 1, g)
            gn = jnp.minimum(g + 1, NGRP - 1)
            xvn = xr[pl.ds(gn * GRP, GRP)]
            fire(xvn, 0, NSW & 1)
            drain((NSW - 1) & 1)
            extract(xv, (NSW - 1) * SUB, (NSW - 1) & 1, g)
            return carry

        xv0 = xr[pl.ds(0, GRP)]
        fire(xv0, 0, 0)
        lax.fori_loop(0, NGRP, group, 0)
        drain(NSW & 1)
        pltpu.sync_copy(ob, out_hbm.at[pl.ds(wid * (BPW // 2), BPW // 2)])

    return body(X, tab_t, shared_flat)


def kernel(X, embed_table, shared_embed):
    out_p = _sc_embed_lookup(X, embed_table.T, shared_embed.reshape(D))
    return out_p.reshape(B, D)
